# speedup vs baseline: 1.0337x; 1.0337x over previous
import functools

import jax
import jax.numpy as jnp
from jax import lax
from jax.experimental import pallas as pl
from jax.experimental.pallas import tpu as pltpu
from jax.experimental.pallas import tpu_sc as plsc

_INFO = plsc.get_sparse_core_info()
_NC, _NS = _INFO.num_cores, _INFO.num_subcores
_NW = _NC * _NS


@functools.cache
def _build(B, V, D, rows_per_worker):
    n_active = B // rows_per_worker
    mesh = plsc.VectorSubcoreMesh(core_axis_name="c", subcore_axis_name="s")

    @functools.partial(
        pl.kernel,
        out_type=jax.ShapeDtypeStruct((B, D), jnp.float32),
        mesh=mesh,
        scratch_types=[
            pltpu.VMEM((rows_per_worker,), jnp.int32),
            pltpu.VMEM((rows_per_worker, D), jnp.float32),
            pltpu.SemaphoreType.DMA,
        ],
    )
    def gather_kernel(idx_hbm, table_hbm, out_hbm, idx_v, rows_v, sem):
        wid = lax.axis_index("s") * _NC + lax.axis_index("c")

        @pl.when(wid < n_active)
        def _():
            base = wid * rows_per_worker
            pltpu.sync_copy(idx_hbm.at[pl.ds(base, rows_per_worker)], idx_v)
            pltpu.async_copy(table_hbm.at[idx_v], rows_v, sem).wait()
            pltpu.sync_copy(rows_v, out_hbm.at[pl.ds(base, rows_per_worker)])

    return gather_kernel


def kernel(indices, embedding):
    batch, t = indices.shape
    v, d = embedding.shape
    b = batch * t
    idx_flat = indices.reshape(b).astype(jnp.int32)
    rows_per_worker = 16
    out = _build(b, v, d, rows_per_worker)(idx_flat, embedding)
    return out.reshape(batch, t, d)


# speedup vs baseline: 1.1177x; 1.0813x over previous
import functools

import jax
import jax.numpy as jnp
from jax import lax
from jax.experimental import pallas as pl
from jax.experimental.pallas import tpu as pltpu
from jax.experimental.pallas import tpu_sc as plsc

_INFO = plsc.get_sparse_core_info()
_NC, _NS = _INFO.num_cores, _INFO.num_subcores
_NW = _NC * _NS

_CHUNK = 16
_TAIL = 8


@functools.cache
def _build(batch, t, t_pad, v, d):
    n_full = t // _CHUNK
    tail_valid = t - n_full * _CHUNK
    mesh = plsc.VectorSubcoreMesh(core_axis_name="c", subcore_axis_name="s")

    @functools.partial(
        pl.kernel,
        out_type=jax.ShapeDtypeStruct((batch, t, d), jnp.float32),
        mesh=mesh,
        scratch_types=[
            pltpu.VMEM((_CHUNK,), jnp.int32),
            pltpu.VMEM((_CHUNK, d), jnp.float32),
            pltpu.VMEM((_TAIL,), jnp.int32),
            pltpu.VMEM((_TAIL, d), jnp.float32),
            pltpu.SemaphoreType.DMA,
        ],
    )
    def gather_kernel(idx_hbm, table_hbm, out_hbm, idx_v, rows_v, idx_t, rows_t, sem):
        wid = lax.axis_index("s") * _NC + lax.axis_index("c")

        @pl.when(wid < batch * n_full)
        def _full():
            bi = wid // n_full
            start = (wid % n_full) * _CHUNK
            pltpu.sync_copy(idx_hbm.at[pl.ds(bi * t_pad + start, _CHUNK)], idx_v)
            pltpu.async_copy(table_hbm.at[idx_v], rows_v, sem).wait()
            pltpu.sync_copy(rows_v, out_hbm.at[bi, pl.ds(start, _CHUNK)])

        if tail_valid:

            @pl.when(
                (wid >= batch * n_full) & (wid < batch * n_full + batch)
            )
            def _tail():
                bi = wid - batch * n_full
                start = n_full * _CHUNK
                pltpu.sync_copy(idx_hbm.at[pl.ds(bi * t_pad + start, _TAIL)], idx_t)
                pltpu.async_copy(table_hbm.at[idx_t], rows_t, sem).wait()
                pltpu.sync_copy(
                    rows_t.at[pl.ds(0, tail_valid)],
                    out_hbm.at[bi, pl.ds(start, tail_valid)],
                )

    return gather_kernel


def kernel(indices, embedding):
    batch, t = indices.shape
    v, d = embedding.shape
    t_pad = (t + _TAIL - 1) // _TAIL * _TAIL
    idx = jnp.pad(indices.astype(jnp.int32), ((0, 0), (0, t_pad - t)))
    out = _build(batch, t, t_pad, v, d)(idx.reshape(batch * t_pad), embedding)
    return out
